# baseline (speedup 1.0000x reference)
import functools

import jax
import jax.numpy as jnp
from jax import lax
from jax.experimental import pallas as pl
from jax.experimental.pallas import tpu as pltpu
from jax.experimental.pallas import tpu_sc as plsc

B, T, C, K = 8, 1024, 128, 3
MIN_P, MAX_P = 16, 64
P_MAX = MAX_P
CYC_MAX = (T + MIN_P - 1) // MIN_P
KER = 9
GSIZE = 4
NEG = float(jnp.finfo(jnp.float32).min) / 8.0
KC = K * CYC_MAX
RTAIL = T - 128
SPB = 2


def _ac_body(x_ref, r_ref):
    BLK = 64
    diag = (jax.lax.broadcasted_iota(jnp.int32, (BLK, BLK + MAX_P), 1) -
            jax.lax.broadcasted_iota(jnp.int32, (BLK, BLK + MAX_P), 0))
    lanei = jax.lax.broadcasted_iota(jnp.int32, (1, BLK + MAX_P), 1)
    inband = (lanei >= MIN_P) & (lanei <= MAX_P)
    for i in range(SPB):
        xb = x_ref[i]
        xp = jnp.concatenate([xb, xb[:MAX_P, :]], axis=0)
        gaccs = [jnp.zeros((BLK, BLK + MAX_P), jnp.float32) for _ in range(4)]
        for q in range(T // BLK):
            gaccs[q % 4] = gaccs[q % 4] + jax.lax.dot_general(
                xb[q * BLK:(q + 1) * BLK, :],
                xp[q * BLK:q * BLK + BLK + MAX_P, :],
                (((1,), (1,)), ((), ())), preferred_element_type=jnp.float32)
        gsum = (gaccs[0] + gaccs[1]) + (gaccs[2] + gaccs[3])
        row = jnp.full((1, BLK + MAX_P), NEG, jnp.float32)
        for lag in range(MIN_P, MAX_P + 1):
            v = jnp.sum(jnp.where(diag == lag, gsum, 0.0)) * (1.0 / C)
            row = jnp.where(lanei == lag, v, row)
        r_ref[0, i] = jnp.where(inband, row, NEG)[0]


def _autocorr(x):
    return pl.pallas_call(
        _ac_body,
        grid=(B // SPB,),
        in_specs=[pl.BlockSpec((SPB, T, C), lambda b: (b, 0, 0))],
        out_specs=pl.BlockSpec((1, SPB, 128), lambda b: (b, 0, 0)),
        out_shape=jax.ShapeDtypeStruct((B // SPB, SPB, 128), jnp.float32),
    )(x)


def _topk_sc_body(r_hbm, p_out, w_out, rv, sv):
    cid = lax.axis_index("c")
    sid = lax.axis_index("s")
    wid = sid * 2 + cid

    @pl.when(wid == 0)
    def _():
        pltpu.sync_copy(r_hbm, rv)
        negv = jnp.full((16,), jnp.float32(NEG))
        win_v = []
        win_l = []
        for rnd in range(K):
            cv = negv
            cl = jnp.zeros((16,), jnp.int32)
            for lag in range(MIN_P, MAX_P + 1):
                lagc = jnp.full((16,), lag, jnp.int32)
                v = rv[lag]
                for prev in win_l:
                    v = jnp.where(lagc == prev, negv, v)
                keep = cv >= v
                cv = jnp.where(keep, cv, v)
                cl = jnp.where(keep, cl, lagc)
            win_v.append(cv)
            win_l.append(cl)
        evs = [jnp.exp(v - win_v[0]) for v in win_v]
        es = (evs[0] + evs[1]) + evs[2]
        for rnd in range(K):
            sv[...] = win_l[rnd].astype(jnp.float32)
            pltpu.sync_copy(sv, p_out.at[pl.ds(rnd * 16, 16)])
            sv[...] = evs[rnd] / es
            pltpu.sync_copy(sv, w_out.at[pl.ds(rnd * 16, 16)])


@functools.lru_cache(maxsize=None)
def _make_topk_sc():
    @functools.partial(
        pl.kernel,
        mesh=plsc.VectorSubcoreMesh(core_axis_name="c", subcore_axis_name="s"),
        out_type=[jax.ShapeDtypeStruct((K * 16,), jnp.float32),
                  jax.ShapeDtypeStruct((K * 16,), jnp.float32)],
        scratch_types=[pltpu.VMEM((128, 16), jnp.float32),
                       pltpu.VMEM((16,), jnp.float32)],
    )
    def _topk_sc(r_hbm, p_out, w_out, rv, sv):
        _topk_sc_body(r_hbm, p_out, w_out, rv, sv)

    return _topk_sc


def _dense_body(pw_s, ww_s, x_ref, dwT_ref, pw_ref, gng_ref, gnb_ref,
                gate_ref, rg_ref, out_ref):
    bprog = pl.program_id(0)
    for i in range(SPB):
        row = bprog * SPB + i
        top_p = [pw_s[k, row] for k in range(K)]
        ws = [ww_s[k, row] for k in range(K)]
        _dense_one(x_ref[i], top_p, ws, dwT_ref, pw_ref, gng_ref, gnb_ref,
                   gate_ref, rg_ref, out_ref, i)


def _dense_one(xb, top_p, ws, dwT_ref, pw_ref, gng_ref, gnb_ref, gate_ref,
               rg_ref, out_ref, i):
    recips = [1.0 / p for p in top_p]
    ncycs = [jnp.floor((jnp.float32(T) - 0.5) * r) + 1.0 for r in recips]
    tpads = [n * p for n, p in zip(ncycs, top_p)]

    gng = gng_ref[...]
    gnb = gnb_ref[...]
    rg = rg_ref[...]
    ci = jax.lax.broadcasted_iota(jnp.int32, (C, C), 0).astype(jnp.float32)
    cj = jax.lax.broadcasted_iota(jnp.int32, (C, C), 1).astype(jnp.float32)
    gmat = (jnp.floor(ci * (1.0 / GSIZE)) ==
            jnp.floor(cj * (1.0 / GSIZE))).astype(jnp.float32)

    recip_col = jnp.concatenate(
        [jnp.full((CYC_MAX, 1), r, jnp.float32) for r in recips], axis=0)
    tpad_col = jnp.concatenate(
        [jnp.full((CYC_MAX, 1), t, jnp.float32) for t in tpads], axis=0)

    tW = jax.lax.broadcasted_iota(jnp.int32, (KC, T), 1).astype(jnp.float32)
    cycW = (jax.lax.broadcasted_iota(jnp.int32, (KC, T), 0) &
            (CYC_MAX - 1)).astype(jnp.float32)
    fd1 = jnp.floor((tW + 0.5) * recip_col)
    w1 = jnp.where((fd1 == cycW) & (tW < tpad_col), 1.0, 0.0)
    jW = jax.lax.broadcasted_iota(jnp.int32, (KC, 128), 1).astype(jnp.float32)
    cycWs = (jax.lax.broadcasted_iota(jnp.int32, (KC, 128), 0) &
             (CYC_MAX - 1)).astype(jnp.float32)
    s2 = jnp.float32(2 * (T - 1) - RTAIL) - jW
    fd2 = jnp.floor((s2 + 0.5) * recip_col)
    w2 = jnp.where((fd2 == cycWs) & (s2 >= jnp.float32(T)) & (s2 < tpad_col),
                   1.0, 0.0)
    ustack = (jax.lax.dot(w1, xb, preferred_element_type=jnp.float32) +
              jax.lax.dot(w2, xb[RTAIL:, :],
                          preferred_element_type=jnp.float32)) * (1.0 / P_MAX)

    ub8 = jnp.concatenate(
        [jnp.sum(ustack[k * CYC_MAX:(k + 1) * CYC_MAX, :], axis=0,
                 keepdims=True) * (1.0 / CYC_MAX) for k in range(K)] +
        [jnp.zeros((8 - K, C), jnp.float32)], axis=0)
    gg = jax.lax.dot_general(ub8, gate_ref[...], (((1,), (1,)), ((), ())),
                             preferred_element_type=jnp.float32)
    grows = [1.0 / (1.0 + jnp.exp(-gg[k:k + 1, :])) for k in range(K)]

    z4 = jnp.zeros((KER // 2, C), jnp.float32)
    xc1s = []
    for k in range(K):
        up = jnp.concatenate(
            [z4, ustack[k * CYC_MAX:(k + 1) * CYC_MAX, :], z4], axis=0)
        xc1 = dwT_ref[0:1, :] * up[0:CYC_MAX, :]
        for j in range(1, KER):
            xc1 = xc1 + dwT_ref[j:j + 1, :] * up[j:j + CYC_MAX, :]
        xc1s.append(xc1)
    xc1 = jnp.concatenate(xc1s, axis=0)

    xc2 = jax.lax.dot_general(xc1, pw_ref[...], (((1,), (1,)), ((), ())),
                              preferred_element_type=jnp.float32)
    m1 = jax.lax.dot(xc2, gmat, preferred_element_type=jnp.float32)
    m2 = jax.lax.dot(xc2 * xc2, gmat, preferred_element_type=jnp.float32)
    denom = 1.0 / (GSIZE * CYC_MAX)
    mks = []
    for k in range(K):
        sl = slice(k * CYC_MAX, (k + 1) * CYC_MAX)
        mu = jnp.sum(m1[sl, :], axis=0, keepdims=True) * denom
        var = jnp.sum(m2[sl, :], axis=0, keepdims=True) * denom - mu * mu
        xn = (xc2[sl, :] - mu) * jax.lax.rsqrt(var + 1e-5) * gng + gnb
        xg = 0.5 * xn * (1.0 + jax.lax.erf(xn * 0.7071067811865476))
        mks.append(xg * (grows[k] * rg * ws[k]))
    mks = jnp.concatenate(mks, axis=0)

    tE = jax.lax.broadcasted_iota(jnp.int32, (T, KC), 0).astype(jnp.float32)
    cycE = (jax.lax.broadcasted_iota(jnp.int32, (T, KC), 1) &
            (CYC_MAX - 1)).astype(jnp.float32)
    recip_row = jnp.concatenate(
        [jnp.full((1, CYC_MAX), r, jnp.float32) for r in recips], axis=1)
    fde = jnp.floor((tE + 0.5) * recip_row)
    emat = jnp.where(fde == cycE, 1.0, 0.0)
    acc = jax.lax.dot(emat, mks, preferred_element_type=jnp.float32)

    wsum = (ws[0] + ws[1]) + ws[2]
    out_ref[i] = xb * (1.0 + wsum) + acc


def _dense(x, pvals, wvals, dwT, pw_w, gng, gnb, gate_w, rg):
    grid_spec = pltpu.PrefetchScalarGridSpec(
        num_scalar_prefetch=2,
        grid=(B // SPB,),
        in_specs=[
            pl.BlockSpec((SPB, T, C), lambda b, *_: (b, 0, 0)),
            pl.BlockSpec((KER, C), lambda b, *_: (0, 0)),
            pl.BlockSpec((C, C), lambda b, *_: (0, 0)),
            pl.BlockSpec((1, C), lambda b, *_: (0, 0)),
            pl.BlockSpec((1, C), lambda b, *_: (0, 0)),
            pl.BlockSpec((C, C), lambda b, *_: (0, 0)),
            pl.BlockSpec((1, C), lambda b, *_: (0, 0)),
        ],
        out_specs=pl.BlockSpec((SPB, T, C), lambda b, *_: (b, 0, 0)),
    )
    return pl.pallas_call(
        _dense_body,
        grid_spec=grid_spec,
        out_shape=jax.ShapeDtypeStruct((B, T, C), jnp.float32),
    )(pvals, wvals, x, dwT, pw_w, gng, gnb, gate_w, rg)


@jax.jit
def kernel(x, dw_w, pw_w, gn_g, gn_b, gate_w, res_g):
    dwT = jnp.transpose(dw_w[:, 0, :], (1, 0))
    gng = gn_g.reshape(1, C)
    gnb = gn_b.reshape(1, C)
    rg = res_g.reshape(1, C)
    r = _autocorr(x).reshape(B, 128)
    r_t = jnp.concatenate(
        [jnp.transpose(r), jnp.full((128, 8), NEG, jnp.float32)], axis=1)
    pvals, wvals = _make_topk_sc()(r_t)
    pvals = pvals.reshape(K, 16)
    wvals = wvals.reshape(K, 16)
    return _dense(x, pvals, wvals, dwT, pw_w, gng, gnb, gate_w, rg)
